# Initial kernel scaffold; baseline (speedup 1.0000x reference)
import jax
import jax.numpy as jnp
from jax.experimental import pallas as pl


def kernel(x, edge_index, W1, b1, W2, b2):
    raise NotImplementedError("write your pallas kernel here")


# speedup vs baseline: 12.0816x; 12.0816x over previous
import functools

import jax
import jax.numpy as jnp
from jax import lax
from jax.experimental import pallas as pl
from jax.experimental.pallas import tpu as pltpu
from jax.experimental.pallas import tpu_sc as plsc

N = 10000
NP = 10240
D_IN = 128
D_HID = 256
D_OUT = 128
E = 320000
C = 128
NC = 2
NS = 16
NW = NC * NS
G = -(-E // (NW * C))
EP = G * NW * C
ROWS_PER_TILE = NP // NS

_mesh = plsc.VectorSubcoreMesh(core_axis_name="c", subcore_axis_name="s")


def _deg_body(dstm, out, didx, onev, zvec, acc):
    cid = lax.axis_index("c")
    sid = lax.axis_index("s")
    wid = sid * NC + cid
    for i in range(8):
        onev[pl.ds(i * 16, 16)] = jnp.full((16,), 1.0, jnp.float32)
    for i in range(ROWS_PER_TILE // 16):
        zvec[pl.ds(i * 16, 16)] = jnp.zeros((16,), jnp.float32)
    pltpu.sync_copy(zvec, acc.at[pl.ds(sid * ROWS_PER_TILE, ROWS_PER_TILE)])
    plsc.subcore_barrier()

    def step(g, carry):
        row = wid * G + g
        pltpu.sync_copy(dstm.at[row], didx)
        pltpu.sync_copy(onev, acc.at[didx], add=True)
        return carry

    lax.fori_loop(0, G, step, 0)
    plsc.subcore_barrier()
    pltpu.sync_copy(
        acc.at[pl.ds(sid * ROWS_PER_TILE, ROWS_PER_TILE)],
        out.at[cid, pl.ds(sid * ROWS_PER_TILE, ROWS_PER_TILE)],
    )


_deg_call = functools.partial(
    pl.kernel,
    out_type=jax.ShapeDtypeStruct((NC, NP), jnp.float32),
    mesh=_mesh,
    scratch_types=[
        pltpu.VMEM((C,), jnp.int32),
        pltpu.VMEM((C,), jnp.float32),
        pltpu.VMEM((ROWS_PER_TILE,), jnp.float32),
        pltpu.VMEM_SHARED((NP,), jnp.float32),
    ],
)(_deg_body)


def _agg_body(table, srcm, dstm, zeros2d, out, sidx, didx, rows, zbuf, acc, sem):
    cid = lax.axis_index("c")
    sid = lax.axis_index("s")
    wid = sid * NC + cid
    pltpu.sync_copy(zeros2d, zbuf)
    for k in range(ROWS_PER_TILE // C):
        pltpu.sync_copy(zbuf, acc.at[pl.ds(sid * ROWS_PER_TILE + k * C, C)])
    plsc.subcore_barrier()

    def step(g, carry):
        row = wid * G + g
        pltpu.sync_copy(srcm.at[row], sidx)
        pltpu.sync_copy(dstm.at[row], didx)
        pltpu.async_copy(table.at[sidx], rows, sem).wait()
        pltpu.sync_copy(rows, acc.at[didx], add=True)
        return carry

    lax.fori_loop(0, G, step, 0)
    plsc.subcore_barrier()
    pltpu.sync_copy(
        acc.at[pl.ds(sid * ROWS_PER_TILE, ROWS_PER_TILE)],
        out.at[cid, pl.ds(sid * ROWS_PER_TILE, ROWS_PER_TILE)],
    )


_agg_call = functools.partial(
    pl.kernel,
    out_type=jax.ShapeDtypeStruct((NC, NP, D_IN), jnp.float32),
    mesh=_mesh,
    scratch_types=[
        pltpu.VMEM((C,), jnp.int32),
        pltpu.VMEM((C,), jnp.int32),
        pltpu.VMEM((C, D_IN), jnp.float32),
        pltpu.VMEM((C, D_IN), jnp.float32),
        pltpu.VMEM_SHARED((NP, D_IN), jnp.float32),
        pltpu.SemaphoreType.DMA,
    ],
)(_agg_body)


ROW_BLK = 512
_GRID = (NP // ROW_BLK,)


def _scale_body(d0, d1, x, s_out, xs_out):
    s = lax.rsqrt(d0[...] + d1[...] + 1.0)
    s_out[...] = s
    xs_out[...] = x[...] * s


_scale_call = pl.pallas_call(
    _scale_body,
    grid=_GRID,
    in_specs=[
        pl.BlockSpec((ROW_BLK, 1), lambda i: (i, 0)),
        pl.BlockSpec((ROW_BLK, 1), lambda i: (i, 0)),
        pl.BlockSpec((ROW_BLK, D_IN), lambda i: (i, 0)),
    ],
    out_specs=[
        pl.BlockSpec((ROW_BLK, 1), lambda i: (i, 0)),
        pl.BlockSpec((ROW_BLK, D_IN), lambda i: (i, 0)),
    ],
    out_shape=[
        jax.ShapeDtypeStruct((NP, 1), jnp.float32),
        jax.ShapeDtypeStruct((NP, D_IN), jnp.float32),
    ],
)


def _layer_body(a0, a1, xs, s, w1, b1, w2, gs_out):
    z = (a0[...] + a1[...] + xs[...]) * s[...]
    h = jnp.dot(z, w1[...], preferred_element_type=jnp.float32) + b1[...]
    h = jnp.maximum(h, 0.0)
    g = jnp.dot(h, w2[...], preferred_element_type=jnp.float32)
    gs_out[...] = g * s[...]


_layer_call = pl.pallas_call(
    _layer_body,
    grid=_GRID,
    in_specs=[
        pl.BlockSpec((ROW_BLK, D_IN), lambda i: (i, 0)),
        pl.BlockSpec((ROW_BLK, D_IN), lambda i: (i, 0)),
        pl.BlockSpec((ROW_BLK, D_IN), lambda i: (i, 0)),
        pl.BlockSpec((ROW_BLK, 1), lambda i: (i, 0)),
        pl.BlockSpec((D_IN, D_HID), lambda i: (0, 0)),
        pl.BlockSpec((1, D_HID), lambda i: (0, 0)),
        pl.BlockSpec((D_HID, D_OUT), lambda i: (0, 0)),
    ],
    out_specs=pl.BlockSpec((ROW_BLK, D_OUT), lambda i: (i, 0)),
    out_shape=jax.ShapeDtypeStruct((NP, D_OUT), jnp.float32),
)


def _final_body(a0, a1, gs, s, b2, o_out):
    o_out[...] = (a0[...] + a1[...] + gs[...]) * s[...] + b2[...]


_final_call = pl.pallas_call(
    _final_body,
    grid=_GRID,
    in_specs=[
        pl.BlockSpec((ROW_BLK, D_OUT), lambda i: (i, 0)),
        pl.BlockSpec((ROW_BLK, D_OUT), lambda i: (i, 0)),
        pl.BlockSpec((ROW_BLK, D_OUT), lambda i: (i, 0)),
        pl.BlockSpec((ROW_BLK, 1), lambda i: (i, 0)),
        pl.BlockSpec((1, D_OUT), lambda i: (0, 0)),
    ],
    out_specs=pl.BlockSpec((ROW_BLK, D_OUT), lambda i: (i, 0)),
    out_shape=jax.ShapeDtypeStruct((NP, D_OUT), jnp.float32),
)


def kernel(x, edge_index, W1, b1, W2, b2):
    ei = edge_index.astype(jnp.int32)
    pad_e = EP - E
    src = jnp.concatenate([ei[0], jnp.zeros((pad_e,), jnp.int32)])
    dst = jnp.concatenate([ei[1], jnp.full((pad_e,), N, jnp.int32)])
    srcm = src.reshape(EP // C, C)
    dstm = dst.reshape(EP // C, C)
    xp = jnp.pad(x, ((0, NP - N), (0, 0)))
    zeros2d = jnp.zeros((C, D_IN), jnp.float32)

    deg = _deg_call(dstm)
    s, xs = _scale_call(deg[0].reshape(NP, 1), deg[1].reshape(NP, 1), xp)
    agg1 = _agg_call(xs, srcm, dstm, zeros2d)
    gs = _layer_call(agg1[0], agg1[1], xs, s,
                     W1, b1.reshape(1, D_HID), W2)
    agg2 = _agg_call(gs, srcm, dstm, zeros2d)
    outp = _final_call(agg2[0], agg2[1], gs, s, b2.reshape(1, D_OUT))
    return outp[:N]
